# Initial kernel scaffold; baseline (speedup 1.0000x reference)
import jax
import jax.numpy as jnp
from jax.experimental import pallas as pl


def kernel(z, edge_index, w1, b1, w2, b2):
    raise NotImplementedError("write your pallas kernel here")


# speedup vs baseline: 1.0054x; 1.0054x over previous
import jax
import jax.numpy as jnp
from jax.experimental import pallas as pl
from jax.experimental.pallas import tpu as pltpu

_LANES = 128
_NODE_TILE = 4096
_EDGE_TILE_ROWS = 2048


def _ceil_to(x, m):
    return ((x + m - 1) // m) * m


def _scores_kernel(z_ref, w_ref, b_ref, s_ref):
    s_ref[...] = (
        jnp.dot(z_ref[...], w_ref[...], preferred_element_type=jnp.float32)
        + b_ref[...]
    )


def _combine_kernel(g_ref, o_ref):
    o_ref[...] = g_ref[0] + g_ref[1]


def kernel(z, edge_index, w1, b1, w2, b2):
    N, D = z.shape
    E = edge_index.shape[1]

    w_fused = w1 @ w2
    b_fused = (b1 @ w2)[0] + b2[0]
    w_node = jnp.concatenate([w_fused[:D], w_fused[D:]], axis=1)
    bias2 = jnp.stack([b_fused, jnp.zeros_like(b_fused)]).reshape(1, 2)

    tile_n = min(_NODE_TILE, _ceil_to(N, 8))
    n_pad = _ceil_to(N, tile_n)
    z_p = jnp.pad(z, ((0, n_pad - N), (0, 0))) if n_pad != N else z
    s = pl.pallas_call(
        _scores_kernel,
        out_shape=jax.ShapeDtypeStruct((n_pad, 2), jnp.float32),
        grid=(n_pad // tile_n,),
        in_specs=[
            pl.BlockSpec((tile_n, D), lambda i: (i, 0)),
            pl.BlockSpec((D, 2), lambda i: (0, 0)),
            pl.BlockSpec((1, 2), lambda i: (0, 0)),
        ],
        out_specs=pl.BlockSpec((tile_n, 2), lambda i: (i, 0)),
        compiler_params=pltpu.CompilerParams(
            dimension_semantics=("parallel",)),
    )(z_p, w_node, bias2)

    s_flat = s[:N].T.reshape(-1)
    idx = edge_index + jnp.array([[0], [N]], dtype=edge_index.dtype)
    g = jnp.take(s_flat, idx, axis=0)

    rows = _ceil_to(E, _LANES) // _LANES
    tile_r = min(_EDGE_TILE_ROWS, _ceil_to(rows, 8))
    rows_pad = _ceil_to(rows, tile_r)
    e_pad = rows_pad * _LANES
    if e_pad != E:
        g = jnp.pad(g, ((0, 0), (0, e_pad - E)))
    g = g.reshape(2, rows_pad, _LANES)

    out = pl.pallas_call(
        _combine_kernel,
        out_shape=jax.ShapeDtypeStruct((rows_pad, _LANES), jnp.float32),
        grid=(rows_pad // tile_r,),
        in_specs=[
            pl.BlockSpec((2, tile_r, _LANES), lambda i: (0, i, 0)),
        ],
        out_specs=pl.BlockSpec((tile_r, _LANES), lambda i: (i, 0)),
        compiler_params=pltpu.CompilerParams(
            dimension_semantics=("parallel",)),
    )(g)

    return out.reshape(-1)[:E].reshape(E, 1)


# speedup vs baseline: 6.2022x; 6.1692x over previous
import jax
import jax.numpy as jnp
from jax.experimental import pallas as pl
from jax.experimental.pallas import tpu as pltpu

_LANES = 128
_NODE_TILE = 4096
_EDGES_PER_STEP = 4096
_CHUNK = 128


def _ceil_to(x, m):
    return ((x + m - 1) // m) * m


def _scores_kernel(z_ref, w_ref, b_ref, s_ref):
    s_ref[...] = (
        jnp.dot(z_ref[...], w_ref[...], preferred_element_type=jnp.float32)
        + b_ref[...]
    )


def _gather_kernel(hia_ref, sha_ref, hib_ref, shb_ref, tbl_ref, o_ref):
    n_chunks = o_ref.shape[0]

    def body(k, carry):
        base = k * _CHUNK
        for j in range(_CHUNK):
            e = base + j
            row_a = tbl_ref[hia_ref[0, 0, e]]
            row_b = tbl_ref[hib_ref[0, 0, e]]
            va = pltpu.roll(row_a, sha_ref[0, 0, e], 1)
            vb = pltpu.roll(row_b, shb_ref[0, 0, e], 1)
            v = va + vb
            o_ref[k, 0, j:j + 1] = v[0, j:j + 1]
        return carry

    jax.lax.fori_loop(0, n_chunks, body, 0)


def kernel(z, edge_index, w1, b1, w2, b2):
    N, D = z.shape
    E = edge_index.shape[1]

    w_fused = w1 @ w2
    b_fused = (b1 @ w2)[0] + b2[0]
    w_node = jnp.concatenate([w_fused[:D], w_fused[D:]], axis=1)
    bias2 = jnp.stack([b_fused, jnp.zeros_like(b_fused)]).reshape(1, 2)

    tile_n = min(_NODE_TILE, _ceil_to(N, 8))
    n_pad = _ceil_to(N, tile_n)
    z_p = jnp.pad(z, ((0, n_pad - N), (0, 0))) if n_pad != N else z
    s = pl.pallas_call(
        _scores_kernel,
        out_shape=jax.ShapeDtypeStruct((n_pad, 2), jnp.float32),
        grid=(n_pad // tile_n,),
        in_specs=[
            pl.BlockSpec((tile_n, D), lambda i: (i, 0)),
            pl.BlockSpec((D, 2), lambda i: (0, 0)),
            pl.BlockSpec((1, 2), lambda i: (0, 0)),
        ],
        out_specs=pl.BlockSpec((tile_n, 2), lambda i: (i, 0)),
        compiler_params=pltpu.CompilerParams(
            dimension_semantics=("parallel",)),
    )(z_p, w_node, bias2)

    half = _ceil_to(N, _LANES)
    s_a = jnp.pad(s[:N, 0], (0, half - N))
    s_b = jnp.pad(s[:N, 1], (0, half - N))
    tbl = jnp.concatenate([s_a, s_b]).reshape(2 * half // _LANES, 1, _LANES)

    e_pad = _ceil_to(E, _EDGES_PER_STEP)
    src = jnp.pad(edge_index[0], (0, e_pad - E))
    dst = jnp.pad(edge_index[1], (0, e_pad - E))
    out_lane = jnp.arange(e_pad, dtype=jnp.int32) % _LANES
    hia = src >> 7
    hib = (dst >> 7) + (half // _LANES)
    sha = (out_lane - (src & (_LANES - 1))) & (_LANES - 1)
    shb = (out_lane - (dst & (_LANES - 1))) & (_LANES - 1)

    n_steps = e_pad // _EDGES_PER_STEP
    idx_shape = (n_steps, 1, _EDGES_PER_STEP)
    hia = hia.reshape(idx_shape)
    hib = hib.reshape(idx_shape)
    sha = sha.reshape(idx_shape)
    shb = shb.reshape(idx_shape)

    rows_step = _EDGES_PER_STEP // _LANES
    idx_spec = pl.BlockSpec((1, 1, _EDGES_PER_STEP), lambda i: (i, 0, 0),
                            memory_space=pltpu.SMEM)
    out = pl.pallas_call(
        _gather_kernel,
        out_shape=jax.ShapeDtypeStruct((n_steps * rows_step, 1, _LANES),
                                       jnp.float32),
        grid=(n_steps,),
        in_specs=[
            idx_spec, idx_spec, idx_spec, idx_spec,
            pl.BlockSpec((2 * half // _LANES, 1, _LANES),
                         lambda i: (0, 0, 0)),
        ],
        out_specs=pl.BlockSpec((rows_step, 1, _LANES), lambda i: (i, 0, 0)),
        compiler_params=pltpu.CompilerParams(
            dimension_semantics=("parallel",)),
    )(hia, sha, hib, shb, tbl)

    return out.reshape(-1)[:E].reshape(E, 1)


# speedup vs baseline: 7.4930x; 1.2081x over previous
import jax
import jax.numpy as jnp
from jax.experimental import pallas as pl
from jax.experimental.pallas import tpu as pltpu

_LANES = 128
_NODE_TILE = 4096
_EDGES_PER_STEP = 8192
_CHUNK = 128


def _ceil_to(x, m):
    return ((x + m - 1) // m) * m


def _scores_kernel(z_ref, w_ref, b_ref, s_ref):
    s_ref[...] = (
        jnp.dot(z_ref[...], w_ref[...], preferred_element_type=jnp.float32)
        + b_ref[...]
    )


def _gather_kernel(hia_ref, hib_ref, loa_ref, lob_ref, tbl_ref, o_ref,
                   ra0, rb0, ra1, rb1):
    n_rows = o_ref.shape[0]
    iota_sub = jax.lax.broadcasted_iota(jnp.int32, (_CHUNK, _LANES), 0)
    eye = jnp.eye(_CHUNK, dtype=jnp.float32)

    def fetch(k, ra, rb):
        for j in range(_CHUNK):
            ra[j] = tbl_ref[hia_ref[k, 0, j]][0]
            rb[j] = tbl_ref[hib_ref[k, 0, j]][0]

    def extract(k, ra, rb):
        m2a = jnp.where(iota_sub == loa_ref[k], 1.0, 0.0)
        m2b = jnp.where(iota_sub == lob_ref[k], 1.0, 0.0)
        p = (jnp.dot(ra[...], m2a, preferred_element_type=jnp.float32)
             + jnp.dot(rb[...], m2b, preferred_element_type=jnp.float32))
        o_ref[k, 0, :] = jnp.sum(p * eye, axis=0)

    def body(i, carry):
        k0 = i * 2
        fetch(k0, ra0, rb0)
        fetch(k0 + 1, ra1, rb1)
        extract(k0, ra0, rb0)
        extract(k0 + 1, ra1, rb1)
        return carry

    jax.lax.fori_loop(0, n_rows // 2, body, 0)


def kernel(z, edge_index, w1, b1, w2, b2):
    N, D = z.shape
    E = edge_index.shape[1]

    w_fused = w1 @ w2
    b_fused = (b1 @ w2)[0] + b2[0]
    w_node = jnp.concatenate([w_fused[:D], w_fused[D:]], axis=1)
    bias2 = jnp.stack([b_fused, jnp.zeros_like(b_fused)]).reshape(1, 2)

    tile_n = min(_NODE_TILE, _ceil_to(N, 8))
    n_pad = _ceil_to(N, tile_n)
    z_p = jnp.pad(z, ((0, n_pad - N), (0, 0))) if n_pad != N else z
    s = pl.pallas_call(
        _scores_kernel,
        out_shape=jax.ShapeDtypeStruct((n_pad, 2), jnp.float32),
        grid=(n_pad // tile_n,),
        in_specs=[
            pl.BlockSpec((tile_n, D), lambda i: (i, 0)),
            pl.BlockSpec((D, 2), lambda i: (0, 0)),
            pl.BlockSpec((1, 2), lambda i: (0, 0)),
        ],
        out_specs=pl.BlockSpec((tile_n, 2), lambda i: (i, 0)),
        compiler_params=pltpu.CompilerParams(
            dimension_semantics=("parallel",)),
    )(z_p, w_node, bias2)

    half = _ceil_to(N, _LANES)
    s_a = jnp.pad(s[:N, 0], (0, half - N))
    s_b = jnp.pad(s[:N, 1], (0, half - N))
    tbl = jnp.concatenate([s_a, s_b]).reshape(2 * half // _LANES, 1, _LANES)

    e_pad = _ceil_to(E, _EDGES_PER_STEP)
    src = jnp.pad(edge_index[0], (0, e_pad - E))
    dst = jnp.pad(edge_index[1], (0, e_pad - E))
    hia = (src >> 7).reshape(-1, 1, _LANES)
    hib = ((dst >> 7) + (half // _LANES)).reshape(-1, 1, _LANES)
    loa = (src & (_LANES - 1)).reshape(-1, 1, _LANES)
    lob = (dst & (_LANES - 1)).reshape(-1, 1, _LANES)

    n_steps = e_pad // _EDGES_PER_STEP
    rows_step = _EDGES_PER_STEP // _LANES
    idx_spec = pl.BlockSpec((rows_step, 1, _LANES), lambda i: (i, 0, 0),
                            memory_space=pltpu.SMEM)
    lo_spec = pl.BlockSpec((rows_step, 1, _LANES), lambda i: (i, 0, 0))
    out = pl.pallas_call(
        _gather_kernel,
        out_shape=jax.ShapeDtypeStruct((n_steps * rows_step, 1, _LANES),
                                       jnp.float32),
        grid=(n_steps,),
        in_specs=[
            idx_spec, idx_spec, lo_spec, lo_spec,
            pl.BlockSpec((2 * half // _LANES, 1, _LANES),
                         lambda i: (0, 0, 0)),
        ],
        out_specs=pl.BlockSpec((rows_step, 1, _LANES), lambda i: (i, 0, 0)),
        scratch_shapes=[pltpu.VMEM((_CHUNK, _LANES), jnp.float32)
                        for _ in range(4)],
        compiler_params=pltpu.CompilerParams(
            dimension_semantics=("parallel",)),
    )(hia, hib, loa, lob, tbl)

    return out.reshape(-1)[:E].reshape(E, 1)


# speedup vs baseline: 7.4936x; 1.0001x over previous
import jax
import jax.numpy as jnp
from jax.experimental import pallas as pl
from jax.experimental.pallas import tpu as pltpu

_LANES = 128
_NODE_TILE = 4096
_EDGES_PER_STEP = 32768
_CHUNK = 128


def _ceil_to(x, m):
    return ((x + m - 1) // m) * m


def _scores_kernel(z_ref, w_ref, b_ref, s_ref):
    s_ref[...] = (
        jnp.dot(z_ref[...], w_ref[...], preferred_element_type=jnp.float32)
        + b_ref[...]
    )


def _gather_kernel(hia_ref, hib_ref, loa_ref, lob_ref, tbl_ref, o_ref,
                   ra0, rb0, ra1, rb1):
    n_rows = o_ref.shape[0]
    iota_sub = jax.lax.broadcasted_iota(jnp.int32, (_CHUNK, _LANES), 0)
    eye = jnp.eye(_CHUNK, dtype=jnp.float32)

    def fetch(k, ra, rb):
        for j in range(_CHUNK):
            ra[j] = tbl_ref[hia_ref[k, 0, j]][0]
            rb[j] = tbl_ref[hib_ref[k, 0, j]][0]

    def extract(k, ra, rb):
        m2a = jnp.where(iota_sub == loa_ref[k], 1.0, 0.0)
        m2b = jnp.where(iota_sub == lob_ref[k], 1.0, 0.0)
        p = (jnp.dot(ra[...], m2a, preferred_element_type=jnp.float32)
             + jnp.dot(rb[...], m2b, preferred_element_type=jnp.float32))
        o_ref[k, 0, :] = jnp.sum(p * eye, axis=0)

    def body(i, carry):
        k0 = i * 2
        fetch(k0, ra0, rb0)
        fetch(k0 + 1, ra1, rb1)
        extract(k0, ra0, rb0)
        extract(k0 + 1, ra1, rb1)
        return carry

    jax.lax.fori_loop(0, n_rows // 2, body, 0)


def kernel(z, edge_index, w1, b1, w2, b2):
    N, D = z.shape
    E = edge_index.shape[1]

    w_fused = w1 @ w2
    b_fused = (b1 @ w2)[0] + b2[0]
    w_node = jnp.concatenate([w_fused[:D], w_fused[D:]], axis=1)
    bias2 = jnp.stack([b_fused, jnp.zeros_like(b_fused)]).reshape(1, 2)

    tile_n = min(_NODE_TILE, _ceil_to(N, 8))
    n_pad = _ceil_to(N, tile_n)
    z_p = jnp.pad(z, ((0, n_pad - N), (0, 0))) if n_pad != N else z
    s = pl.pallas_call(
        _scores_kernel,
        out_shape=jax.ShapeDtypeStruct((n_pad, 2), jnp.float32),
        grid=(n_pad // tile_n,),
        in_specs=[
            pl.BlockSpec((tile_n, D), lambda i: (i, 0)),
            pl.BlockSpec((D, 2), lambda i: (0, 0)),
            pl.BlockSpec((1, 2), lambda i: (0, 0)),
        ],
        out_specs=pl.BlockSpec((tile_n, 2), lambda i: (i, 0)),
        compiler_params=pltpu.CompilerParams(
            dimension_semantics=("parallel",)),
    )(z_p, w_node, bias2)

    half = _ceil_to(N, _LANES)
    s_a = jnp.pad(s[:N, 0], (0, half - N))
    s_b = jnp.pad(s[:N, 1], (0, half - N))
    tbl = jnp.concatenate([s_a, s_b]).reshape(2 * half // _LANES, 1, _LANES)

    e_pad = _ceil_to(E, _EDGES_PER_STEP)
    src = jnp.pad(edge_index[0], (0, e_pad - E))
    dst = jnp.pad(edge_index[1], (0, e_pad - E))
    hia = (src >> 7).reshape(-1, 1, _LANES)
    hib = ((dst >> 7) + (half // _LANES)).reshape(-1, 1, _LANES)
    loa = (src & (_LANES - 1)).reshape(-1, 1, _LANES)
    lob = (dst & (_LANES - 1)).reshape(-1, 1, _LANES)

    n_steps = e_pad // _EDGES_PER_STEP
    rows_step = _EDGES_PER_STEP // _LANES
    idx_spec = pl.BlockSpec((rows_step, 1, _LANES), lambda i: (i, 0, 0),
                            memory_space=pltpu.SMEM)
    lo_spec = pl.BlockSpec((rows_step, 1, _LANES), lambda i: (i, 0, 0))
    out = pl.pallas_call(
        _gather_kernel,
        out_shape=jax.ShapeDtypeStruct((n_steps * rows_step, 1, _LANES),
                                       jnp.float32),
        grid=(n_steps,),
        in_specs=[
            idx_spec, idx_spec, lo_spec, lo_spec,
            pl.BlockSpec((2 * half // _LANES, 1, _LANES),
                         lambda i: (0, 0, 0)),
        ],
        out_specs=pl.BlockSpec((rows_step, 1, _LANES), lambda i: (i, 0, 0)),
        scratch_shapes=[pltpu.VMEM((_CHUNK, _LANES), jnp.float32)
                        for _ in range(4)],
        compiler_params=pltpu.CompilerParams(
            dimension_semantics=("parallel",)),
    )(hia, hib, loa, lob, tbl)

    return out.reshape(-1)[:E].reshape(E, 1)


# speedup vs baseline: 9.3529x; 1.2481x over previous
import jax
import jax.numpy as jnp
from jax.experimental import pallas as pl
from jax.experimental.pallas import tpu as pltpu

_LANES = 128
_NODE_TILE = 4096
_EDGES_PER_STEP = 32768
_CHUNK = 128
_HALF = 512


def _ceil_to(x, m):
    return ((x + m - 1) // m) * m


def _scores_kernel(z_ref, w_ref, b_ref, s_ref):
    s_ref[...] = (
        jnp.dot(z_ref[...], w_ref[...], preferred_element_type=jnp.float32)
        + b_ref[...]
    )


def _gather_kernel(hia_ref, hib_ref, loa_ref, lob_ref, tbl_ref, o_ref,
                   ra0, rb0, ra1, rb1, ra2, rb2, ra3, rb3,
                   sa0, sb0, sa1, sb1, sem0a, sem0b, sem1a, sem1b):
    n_rows = o_ref.shape[0]
    n_pairs = n_rows * _LANES // (2 * _HALF)
    base_h = pl.program_id(0) * (n_rows * _LANES // _HALF)
    iota_sub = jax.lax.broadcasted_iota(jnp.int32, (_CHUNK, _LANES), 0)
    eye = jnp.eye(_CHUNK, dtype=jnp.float32)
    bufs = [(ra0, rb0), (ra1, rb1), (ra2, rb2), (ra3, rb3)]

    def dma(h, sa, sb, sema, semb):
        ca = pltpu.make_async_copy(hia_ref.at[h], sa, sema)
        cb = pltpu.make_async_copy(hib_ref.at[h], sb, semb)
        return ca, cb

    def fetch(c, sa, sb, ra, rb):
        base = c * _CHUNK
        for j in range(_CHUNK):
            ra[j] = tbl_ref[sa[base + j]][0]
            rb[j] = tbl_ref[sb[base + j]][0]

    def extract(k, ra, rb):
        m2a = jnp.where(iota_sub == loa_ref[k], 1.0, 0.0)
        m2b = jnp.where(iota_sub == lob_ref[k], 1.0, 0.0)
        p = (jnp.dot(ra[...], m2a, preferred_element_type=jnp.float32)
             + jnp.dot(rb[...], m2b, preferred_element_type=jnp.float32))
        o_ref[k, 0, :] = jnp.sum(p * eye, axis=0)

    def process(row0, sa, sb):
        fetch(0, sa, sb, ra0, rb0)
        fetch(1, sa, sb, ra1, rb1)
        extract(row0 + 0, ra0, rb0)
        fetch(2, sa, sb, ra2, rb2)
        extract(row0 + 1, ra1, rb1)
        fetch(3, sa, sb, ra3, rb3)
        extract(row0 + 2, ra2, rb2)
        extract(row0 + 3, ra3, rb3)

    for c in dma(base_h, sa0, sb0, sem0a, sem0b):
        c.start()
    for c in dma(base_h + 1, sa1, sb1, sem1a, sem1b):
        c.start()

    def body(t, carry):
        h = base_h + 2 * t
        row0 = t * (2 * _HALF // _LANES)

        for c in dma(h, sa0, sb0, sem0a, sem0b):
            c.wait()
        process(row0, sa0, sb0)

        @pl.when(t + 1 < n_pairs)
        def _():
            for c in dma(h + 2, sa0, sb0, sem0a, sem0b):
                c.start()

        for c in dma(h + 1, sa1, sb1, sem1a, sem1b):
            c.wait()
        process(row0 + _HALF // _LANES, sa1, sb1)

        @pl.when(t + 1 < n_pairs)
        def _():
            for c in dma(h + 3, sa1, sb1, sem1a, sem1b):
                c.start()

        return carry

    jax.lax.fori_loop(0, n_pairs, body, 0)


def kernel(z, edge_index, w1, b1, w2, b2):
    N, D = z.shape
    E = edge_index.shape[1]

    w_fused = w1 @ w2
    b_fused = (b1 @ w2)[0] + b2[0]
    w_node = jnp.concatenate([w_fused[:D], w_fused[D:]], axis=1)
    bias2 = jnp.stack([b_fused, jnp.zeros_like(b_fused)]).reshape(1, 2)

    tile_n = min(_NODE_TILE, _ceil_to(N, 8))
    n_pad = _ceil_to(N, tile_n)
    z_p = jnp.pad(z, ((0, n_pad - N), (0, 0))) if n_pad != N else z
    s = pl.pallas_call(
        _scores_kernel,
        out_shape=jax.ShapeDtypeStruct((n_pad, 2), jnp.float32),
        grid=(n_pad // tile_n,),
        in_specs=[
            pl.BlockSpec((tile_n, D), lambda i: (i, 0)),
            pl.BlockSpec((D, 2), lambda i: (0, 0)),
            pl.BlockSpec((1, 2), lambda i: (0, 0)),
        ],
        out_specs=pl.BlockSpec((tile_n, 2), lambda i: (i, 0)),
        compiler_params=pltpu.CompilerParams(
            dimension_semantics=("parallel",)),
    )(z_p, w_node, bias2)

    half = _ceil_to(N, _LANES)
    s_a = jnp.pad(s[:N, 0], (0, half - N))
    s_b = jnp.pad(s[:N, 1], (0, half - N))
    tbl = jnp.concatenate([s_a, s_b]).reshape(2 * half // _LANES, 1, _LANES)

    e_pad = _ceil_to(E, _EDGES_PER_STEP)
    src = jnp.pad(edge_index[0], (0, e_pad - E))
    dst = jnp.pad(edge_index[1], (0, e_pad - E))
    hia = (src >> 7).reshape(-1, _HALF)
    hib = ((dst >> 7) + (half // _LANES)).reshape(-1, _HALF)
    loa = (src & (_LANES - 1)).reshape(-1, 1, _LANES)
    lob = (dst & (_LANES - 1)).reshape(-1, 1, _LANES)

    n_steps = e_pad // _EDGES_PER_STEP
    rows_step = _EDGES_PER_STEP // _LANES
    lo_spec = pl.BlockSpec((rows_step, 1, _LANES), lambda i: (i, 0, 0))
    out = pl.pallas_call(
        _gather_kernel,
        out_shape=jax.ShapeDtypeStruct((n_steps * rows_step, 1, _LANES),
                                       jnp.float32),
        grid=(n_steps,),
        in_specs=[
            pl.BlockSpec(memory_space=pl.ANY),
            pl.BlockSpec(memory_space=pl.ANY),
            lo_spec, lo_spec,
            pl.BlockSpec((2 * half // _LANES, 1, _LANES),
                         lambda i: (0, 0, 0)),
        ],
        out_specs=pl.BlockSpec((rows_step, 1, _LANES), lambda i: (i, 0, 0)),
        scratch_shapes=(
            [pltpu.VMEM((_CHUNK, _LANES), jnp.float32) for _ in range(8)]
            + [pltpu.SMEM((_HALF,), jnp.int32) for _ in range(4)]
            + [pltpu.SemaphoreType.DMA for _ in range(4)]
        ),
        compiler_params=pltpu.CompilerParams(
            dimension_semantics=("parallel",)),
    )(hia, hib, loa, lob, tbl)

    return out.reshape(-1)[:E].reshape(E, 1)
